# Initial kernel scaffold; baseline (speedup 1.0000x reference)
import jax
import jax.numpy as jnp
from jax.experimental import pallas as pl


def kernel(k_val, v_val, k_cache, v_cache, cache_pos):
    raise NotImplementedError("write your pallas kernel here")


# speedup vs baseline: 10.3805x; 10.3805x over previous
import functools

import jax
import jax.numpy as jnp
from jax import lax
from jax.experimental import pallas as pl
from jax.experimental.pallas import tpu as pltpu
from jax.experimental.pallas import tpu_sc as plsc

B = 16
H = 8
MAX_SEQ = 4096
S = 16
D = 128

NC = 2
NS = 16
NW = NC * NS
ROWS = B * H * S
RPW = ROWS // NW
BPW = RPW // S
DUMP = ROWS


def _body(kv, vv, kc, vc, pos_h, ko, vo, buf, idx, posb, sem):
    wid = lax.axis_index("s") * NC + lax.axis_index("c")
    base = wid * RPW

    pltpu.sync_copy(pos_h.at[pl.ds(0, S)], posb)
    pos = posb[...]
    inb = jnp.logical_and(pos >= 0, pos < S)

    for src, dst in ((kv, ko), (vv, vo)):
        for t in range(BPW):
            bh = wid * BPW + t
            cache = kc if src is kv else vc
            pltpu.sync_copy(cache.at[pl.ds(bh * MAX_SEQ, S)],
                            buf.at[pl.ds(t * S, S)])
        pltpu.sync_copy(buf, dst.at[pl.ds(base, RPW)])

        for t in range(BPW):
            bh = wid * BPW + t
            d = jnp.where(inb, bh * S + pos, DUMP)
            idx[pl.ds(t * S, S)] = d

        pltpu.sync_copy(src.at[pl.ds(base, RPW)], buf)
        pltpu.async_copy(buf, dst.at[idx], sem).wait()


@jax.jit
def _cache_update(kv, vv, kc, vc, cache_pos):
    mesh = plsc.VectorSubcoreMesh(core_axis_name="c", subcore_axis_name="s",
                                  num_cores=NC, num_subcores=NS)
    out = jax.ShapeDtypeStruct((ROWS + 1, D), jnp.float32)
    ko, vo = pl.kernel(
        _body,
        out_type=(out, out),
        mesh=mesh,
        scratch_types=[
            pltpu.VMEM((RPW, D), jnp.float32),
            pltpu.VMEM((RPW,), jnp.int32),
            pltpu.VMEM((S,), jnp.int32),
            pltpu.SemaphoreType.DMA,
        ],
    )(kv, vv, kc, vc, cache_pos)
    return ko, vo


def kernel(k_val, v_val, k_cache, v_cache, cache_pos):
    kv = k_val.reshape(ROWS, D)
    vv = v_val.reshape(ROWS, D)
    kc = k_cache.reshape(B * H * MAX_SEQ, D)
    vc = v_cache.reshape(B * H * MAX_SEQ, D)
    ko, vo = _cache_update(kv, vv, kc, vc, cache_pos)
    k_ret = ko[:ROWS].reshape(B, H, S, D)
    v_ret = vo[:ROWS].reshape(B, H, S, D)
    return (k_ret, v_ret)


# speedup vs baseline: 12.8129x; 1.2343x over previous
import functools

import jax
import jax.numpy as jnp
from jax import lax
from jax.experimental import pallas as pl
from jax.experimental.pallas import tpu as pltpu
from jax.experimental.pallas import tpu_sc as plsc

B = 16
H = 8
MAX_SEQ = 4096
S = 16
D = 128

NC = 2
NS = 16
NW = NC * NS
ROWS = B * H * S
RPW = ROWS // NW
BPW = RPW // S
DUMP = ROWS


def _body(kv, vv, kc, vc, pos_h, ko, vo,
          cbufk, cbufv, sbufk, sbufv, idx, posb,
          sem_pos, sem_cache, sem_stage, sem_out, sem_scat):
    wid = lax.axis_index("s") * NC + lax.axis_index("c")
    base = wid * RPW

    cp_pos = pltpu.async_copy(pos_h.at[pl.ds(0, S)], posb, sem_pos)
    cache_cps = []
    for t in range(BPW):
        bh = wid * BPW + t
        cache_cps.append(pltpu.async_copy(
            kc.at[pl.ds(bh * MAX_SEQ, S)], cbufk.at[pl.ds(t * S, S)],
            sem_cache))
        cache_cps.append(pltpu.async_copy(
            vc.at[pl.ds(bh * MAX_SEQ, S)], cbufv.at[pl.ds(t * S, S)],
            sem_cache))
    st_k = pltpu.async_copy(kv.at[pl.ds(base, RPW)], sbufk, sem_stage)
    st_v = pltpu.async_copy(vv.at[pl.ds(base, RPW)], sbufv, sem_stage)

    cp_pos.wait()
    pos = posb[...]
    inb = jnp.logical_and(pos >= 0, pos < S)
    for t in range(BPW):
        bh = wid * BPW + t
        idx[pl.ds(t * S, S)] = jnp.where(inb, bh * S + pos, DUMP)

    for c in cache_cps:
        c.wait()
    w_k = pltpu.async_copy(cbufk, ko.at[pl.ds(base, RPW)], sem_out)
    w_v = pltpu.async_copy(cbufv, vo.at[pl.ds(base, RPW)], sem_out)
    w_k.wait()
    w_v.wait()

    st_k.wait()
    st_v.wait()
    sc_k = pltpu.async_copy(sbufk, ko.at[idx], sem_scat)
    sc_v = pltpu.async_copy(sbufv, vo.at[idx], sem_scat)
    sc_k.wait()
    sc_v.wait()


@jax.jit
def _cache_update(kv, vv, kc, vc, cache_pos):
    mesh = plsc.VectorSubcoreMesh(core_axis_name="c", subcore_axis_name="s",
                                  num_cores=NC, num_subcores=NS)
    out = jax.ShapeDtypeStruct((ROWS + 1, D), jnp.float32)
    ko, vo = pl.kernel(
        _body,
        out_type=(out, out),
        mesh=mesh,
        scratch_types=[
            pltpu.VMEM((RPW, D), jnp.float32),
            pltpu.VMEM((RPW, D), jnp.float32),
            pltpu.VMEM((RPW, D), jnp.float32),
            pltpu.VMEM((RPW, D), jnp.float32),
            pltpu.VMEM((RPW,), jnp.int32),
            pltpu.VMEM((S,), jnp.int32),
            pltpu.SemaphoreType.DMA,
            pltpu.SemaphoreType.DMA,
            pltpu.SemaphoreType.DMA,
            pltpu.SemaphoreType.DMA,
            pltpu.SemaphoreType.DMA,
        ],
    )(kv, vv, kc, vc, cache_pos)
    return ko, vo


def kernel(k_val, v_val, k_cache, v_cache, cache_pos):
    kv = k_val.reshape(ROWS, D)
    vv = v_val.reshape(ROWS, D)
    kc = k_cache.reshape(B * H * MAX_SEQ, D)
    vc = v_cache.reshape(B * H * MAX_SEQ, D)
    ko, vo = _cache_update(kv, vv, kc, vc, cache_pos)
    k_ret = ko[:ROWS].reshape(B, H, S, D)
    v_ret = vo[:ROWS].reshape(B, H, S, D)
    return (k_ret, v_ret)


# speedup vs baseline: 14.8954x; 1.1625x over previous
import functools

import jax
import jax.numpy as jnp
from jax import lax
from jax.experimental import pallas as pl
from jax.experimental.pallas import tpu as pltpu
from jax.experimental.pallas import tpu_sc as plsc

B = 16
H = 8
MAX_SEQ = 4096
S = 16
D = 128

NC = 2
NS = 16
NW = NC * NS
ROWS = B * H * S
RPW = ROWS // NW
BPW = RPW // S


def _body(kv, vv, kc, vc, pos_h, ko, vo,
          cbufk, cbufv, sbufk, sbufv, idx, posb,
          sem_pos, sem_cache, sem_stage, sem_out, sem_scat):
    wid = lax.axis_index("s") * NC + lax.axis_index("c")
    base = wid * RPW

    st_k = pltpu.async_copy(kv.at[pl.ds(base, RPW)], sbufk, sem_stage)
    st_v = pltpu.async_copy(vv.at[pl.ds(base, RPW)], sbufv, sem_stage)

    pltpu.async_copy(pos_h.at[pl.ds(0, S)], posb, sem_pos).wait()
    pos = posb[...]
    inb = jnp.logical_and(pos >= 0, pos < S)
    pos_safe = jnp.where(inb, pos, 0)
    mask = 0
    for j in range(S):
        pj = pos[j]
        valid = jnp.logical_and(pj >= 0, pj < S)
        bit = jnp.where(valid, lax.shift_left(1, pj), 0)
        mask = lax.bitwise_or(mask, bit)
    allcov = mask == (1 << S) - 1

    @pl.when(allcov)
    def _fast():
        for t in range(BPW):
            bh = wid * BPW + t
            idx[pl.ds(t * S, S)] = bh * S + pos_safe
        st_k.wait()
        st_v.wait()
        sc_k = pltpu.async_copy(sbufk, ko.at[idx], sem_scat)
        sc_v = pltpu.async_copy(sbufv, vo.at[idx], sem_scat)
        sc_k.wait()
        sc_v.wait()

    @pl.when(jnp.logical_not(allcov))
    def _general():
        cps = []
        for t in range(BPW):
            bh = wid * BPW + t
            cps.append(pltpu.async_copy(
                kc.at[pl.ds(bh * MAX_SEQ, S)], cbufk.at[pl.ds(t * S, S)],
                sem_cache))
            cps.append(pltpu.async_copy(
                vc.at[pl.ds(bh * MAX_SEQ, S)], cbufv.at[pl.ds(t * S, S)],
                sem_cache))
        for c in cps:
            c.wait()
        w_k = pltpu.async_copy(cbufk, ko.at[pl.ds(base, RPW)], sem_out)
        w_v = pltpu.async_copy(cbufv, vo.at[pl.ds(base, RPW)], sem_out)
        w_k.wait()
        w_v.wait()
        st_k.wait()
        st_v.wait()
        for j in range(S):
            p_j = pos[j]
            ok_j = jnp.logical_and(p_j >= 0, p_j < S)

            @pl.when(ok_j)
            def _row(p_j=p_j, j=j):
                p_c = jnp.clip(p_j, 0, S - 1)
                for t in range(BPW):
                    bh = wid * BPW + t
                    pltpu.sync_copy(sbufk.at[pl.ds(t * S + j, 1)],
                                    ko.at[pl.ds(bh * S + p_c, 1)])
                    pltpu.sync_copy(sbufv.at[pl.ds(t * S + j, 1)],
                                    vo.at[pl.ds(bh * S + p_c, 1)])


@jax.jit
def _cache_update(kv, vv, kc, vc, cache_pos):
    mesh = plsc.VectorSubcoreMesh(core_axis_name="c", subcore_axis_name="s",
                                  num_cores=NC, num_subcores=NS)
    out = jax.ShapeDtypeStruct((ROWS, D), jnp.float32)
    ko, vo = pl.kernel(
        _body,
        out_type=(out, out),
        mesh=mesh,
        scratch_types=[
            pltpu.VMEM((RPW, D), jnp.float32),
            pltpu.VMEM((RPW, D), jnp.float32),
            pltpu.VMEM((RPW, D), jnp.float32),
            pltpu.VMEM((RPW, D), jnp.float32),
            pltpu.VMEM((RPW,), jnp.int32),
            pltpu.VMEM((S,), jnp.int32),
            pltpu.SemaphoreType.DMA,
            pltpu.SemaphoreType.DMA,
            pltpu.SemaphoreType.DMA,
            pltpu.SemaphoreType.DMA,
            pltpu.SemaphoreType.DMA,
        ],
    )(kv, vv, kc, vc, cache_pos)
    return ko, vo


def kernel(k_val, v_val, k_cache, v_cache, cache_pos):
    kv = k_val.reshape(ROWS, D)
    vv = v_val.reshape(ROWS, D)
    kc = k_cache.reshape(B * H * MAX_SEQ, D)
    vc = v_cache.reshape(B * H * MAX_SEQ, D)
    ko, vo = _cache_update(kv, vv, kc, vc, cache_pos)
    k_ret = ko.reshape(B, H, S, D)
    v_ret = vo.reshape(B, H, S, D)
    return (k_ret, v_ret)
